# baseline (speedup 1.0000x reference)
import functools

import jax
import jax.numpy as jnp
from jax import lax
from jax.experimental import pallas as pl
from jax.experimental.pallas import tpu as pltpu
from jax.experimental.pallas import tpu_sc as plsc

VOCAB_SIZE = 1000
EMBED_DIM = 128
BATCH = 4096
SEQ = 200

NC = 2
NS = 16
NW = NC * NS

B = BATCH * SEQ
B_PER_W = B // NW
ROWS = 128
K = 2
GROUP = K * ROWS
NG = B_PER_W // GROUP
NSLOT = 3


def _make_kernel():
    mesh = plsc.VectorSubcoreMesh(
        core_axis_name="c", subcore_axis_name="s",
        num_cores=NC, num_subcores=NS)

    @functools.partial(
        pl.kernel,
        mesh=mesh,
        out_type=jax.ShapeDtypeStruct((B, EMBED_DIM), jnp.float32),
        scratch_types=[
            pltpu.VMEM((NSLOT, K, ROWS), jnp.int32),
            pltpu.VMEM((NSLOT, GROUP, EMBED_DIM), jnp.float32),
            pltpu.VMEM_SHARED((VOCAB_SIZE, EMBED_DIM), jnp.float32),
            [pltpu.SemaphoreType.DMA] * NSLOT,
            [pltpu.SemaphoreType.DMA] * NSLOT,
            [pltpu.SemaphoreType.DMA] * NSLOT,
        ],
    )
    def emb_kernel(x_hbm, w_hbm, out_hbm, idx_v, rows_v, w_sh,
                   isems, gsems, osems):
        sid = lax.axis_index("s")
        wid = sid * NC + lax.axis_index("c")
        base = wid * B_PER_W

        @pl.when(sid == 0)
        def _stage_table():
            pltpu.sync_copy(w_hbm, w_sh)
        plsc.subcore_barrier()

        def fire_i(g, u):
            pltpu.async_copy(x_hbm.at[wid, g], idx_v.at[u], isems[u])

        def drain_i(g, u):
            pltpu.make_async_copy(
                x_hbm.at[wid, g], idx_v.at[u], isems[u]).wait()

        def fire_g(g, u):
            for t in range(K):
                pltpu.async_copy(
                    w_sh.at[idx_v.at[u, t]],
                    rows_v.at[u, pl.ds(t * ROWS, ROWS)],
                    gsems[u])

        def drain_g(g, u):
            for t in range(K):
                pltpu.make_async_copy(
                    w_sh.at[idx_v.at[u, t]],
                    rows_v.at[u, pl.ds(t * ROWS, ROWS)],
                    gsems[u]).wait()

        def fire_s(g, u):
            pltpu.async_copy(
                rows_v.at[u], out_hbm.at[pl.ds(base + g * GROUP, GROUP)],
                osems[u])

        def drain_s(g, u):
            pltpu.make_async_copy(
                rows_v.at[u], out_hbm.at[pl.ds(base + g * GROUP, GROUP)],
                osems[u]).wait()

        def consume(g, u, has_old=True, has_i=True, has_next=True):
            drain_g(g, u)
            fire_s(g, u)
            if has_i:
                fire_i(g + NSLOT, u)
            nu = (u + 1) % NSLOT
            if has_old:
                drain_s(g + 1 - NSLOT, nu)
            if has_next:
                drain_i(g + 1, nu)
                fire_g(g + 1, nu)

        for g in range(NSLOT):
            fire_i(g, g)
        drain_i(0, 0)
        fire_g(0, 0)
        for g in range(NSLOT - 1):
            consume(g, g, has_old=False)
        consume(NSLOT - 1, NSLOT - 1)

        def body(m, carry):
            for u in range(NSLOT):
                consume(m * NSLOT + u, u)
            return carry

        top = ((NG - NSLOT - 1) // NSLOT) * NSLOT
        lax.fori_loop(1, top // NSLOT, body, 0)

        for g in range(top, NG):
            consume(g, g % NSLOT,
                    has_i=(g + NSLOT < NG),
                    has_next=(g + 1 < NG))
        for g in range(NG - (NSLOT - 1), NG):
            drain_s(g, g % NSLOT)

    return emb_kernel


_emb = _make_kernel()


def kernel(x, weight):
    x4 = x.reshape(NW, NG, K, ROWS)
    out = _emb(x4, weight)
    return out.reshape(BATCH, SEQ, EMBED_DIM)


# speedup vs baseline: 1.0018x; 1.0018x over previous
import functools

import jax
import jax.numpy as jnp
from jax import lax
from jax.experimental import pallas as pl
from jax.experimental.pallas import tpu as pltpu
from jax.experimental.pallas import tpu_sc as plsc

VOCAB_SIZE = 1000
EMBED_DIM = 128
BATCH = 4096
SEQ = 200

NC = 2
NS = 16
NW = NC * NS

B = BATCH * SEQ
B_PER_W = B // NW
ROWS = 128
K = 2
GROUP = K * ROWS
NG = B_PER_W // GROUP
NSLOT = 3


def _make_kernel():
    mesh = plsc.VectorSubcoreMesh(
        core_axis_name="c", subcore_axis_name="s",
        num_cores=NC, num_subcores=NS)

    @functools.partial(
        pl.kernel,
        mesh=mesh,
        out_type=jax.ShapeDtypeStruct((B, EMBED_DIM), jnp.float32),
        scratch_types=[
            pltpu.VMEM((NSLOT, K, ROWS), jnp.int32),
            pltpu.VMEM((NSLOT, GROUP, EMBED_DIM), jnp.float32),
            pltpu.VMEM_SHARED((VOCAB_SIZE, EMBED_DIM), jnp.float32),
            [pltpu.SemaphoreType.DMA] * NSLOT,
            [pltpu.SemaphoreType.DMA] * NSLOT,
            [pltpu.SemaphoreType.DMA] * NSLOT,
        ],
    )
    def emb_kernel(x_hbm, w_hbm, out_hbm, idx_v, rows_v, w_sh,
                   isems, gsems, osems):
        sid = lax.axis_index("s")
        wid = sid * NC + lax.axis_index("c")
        base = wid * B_PER_W

        @pl.when(sid == 0)
        def _stage_table():
            pltpu.sync_copy(w_hbm, w_sh)
        plsc.subcore_barrier()

        def fire_i(g, u):
            pltpu.async_copy(x_hbm.at[wid, g], idx_v.at[u], isems[u])

        def drain_i(g, u):
            pltpu.make_async_copy(
                x_hbm.at[wid, g], idx_v.at[u], isems[u]).wait()

        def fire_g(g, u):
            for t in range(K):
                pltpu.async_copy(
                    w_sh.at[idx_v.at[u, t]],
                    rows_v.at[u, pl.ds(t * ROWS, ROWS)],
                    gsems[u])

        def drain_g(g, u):
            for t in range(K):
                pltpu.make_async_copy(
                    w_sh.at[idx_v.at[u, t]],
                    rows_v.at[u, pl.ds(t * ROWS, ROWS)],
                    gsems[u]).wait()

        def fire_s(g, u):
            pltpu.async_copy(
                rows_v.at[u], out_hbm.at[pl.ds(base + g * GROUP, GROUP)],
                osems[u])

        def drain_s(g, u):
            pltpu.make_async_copy(
                rows_v.at[u], out_hbm.at[pl.ds(base + g * GROUP, GROUP)],
                osems[u]).wait()

        def consume(g, u, has_old=True, has_i=True, has_next=True):
            drain_g(g, u)
            nu = (u + 1) % NSLOT
            if has_old:
                drain_s(g + 1 - NSLOT, nu)
            if has_next:
                drain_i(g + 1, nu)
                fire_g(g + 1, nu)
            fire_s(g, u)
            if has_i:
                fire_i(g + NSLOT, u)

        for g in range(NSLOT):
            fire_i(g, g)
        drain_i(0, 0)
        fire_g(0, 0)
        for g in range(NSLOT - 1):
            consume(g, g, has_old=False)
        consume(NSLOT - 1, NSLOT - 1)

        def body(m, carry):
            for u in range(NSLOT):
                consume(m * NSLOT + u, u)
            return carry

        top = ((NG - NSLOT - 1) // NSLOT) * NSLOT
        lax.fori_loop(1, top // NSLOT, body, 0)

        for g in range(top, NG):
            consume(g, g % NSLOT,
                    has_i=(g + NSLOT < NG),
                    has_next=(g + 1 < NG))
        for g in range(NG - (NSLOT - 1), NG):
            drain_s(g, g % NSLOT)

    return emb_kernel


_emb = _make_kernel()


def kernel(x, weight):
    x4 = x.reshape(NW, NG, K, ROWS)
    out = _emb(x4, weight)
    return out.reshape(BATCH, SEQ, EMBED_DIM)
